# baseline (speedup 1.0000x reference)
import functools

import jax
import jax.numpy as jnp
from jax import lax
from jax.experimental import pallas as pl
from jax.experimental.pallas import tpu as pltpu
from jax.experimental.pallas import tpu_sc as plsc

_LANES = 16
_TABLE_PAD = 128
_NCHUNK = 4
_TC_BLK_ROWS = 512


def _sc_part(elems, energy, lut, tsc, tsh, n_sc):
    mesh = plsc.VectorSubcoreMesh(core_axis_name="c", subcore_axis_name="s")
    n_workers = mesh.num_cores * mesh.num_subcores
    per_w = n_sc // n_workers
    ch = per_w // _NCHUNK
    assert n_sc % n_workers == 0 and per_w % (_NCHUNK * _LANES) == 0

    def body(elems_hbm, energy_hbm, lut_hbm, tsc_hbm, tsh_hbm, out_hbm,
             ev0, ev1, av0, av1, ov0, ov1,
             lut_v, tsc_v, tsh_v, fsc_v, fsh_v,
             se0, se1, sa0, sa1, so0, so1):
        wid = lax.axis_index("s") * mesh.num_cores + lax.axis_index("c")
        base = wid * per_w
        evs, avs, ovs = [ev0, ev1], [av0, av1], [ov0, ov1]
        ses, sas, sos = [se0, se1], [sa0, sa1], [so0, so1]

        def fetch(c):
            b = c % 2
            src = pl.ds(base + c * ch, ch)
            return (pltpu.async_copy(elems_hbm.at[src], evs[b], ses[b]),
                    pltpu.async_copy(energy_hbm.at[src], avs[b], sas[b]))

        in_flight = {0: fetch(0)}

        t = lut_hbm.shape[0]
        pltpu.sync_copy(lut_hbm, lut_v.at[pl.ds(0, t)])
        pltpu.sync_copy(tsc_hbm, tsc_v.at[pl.ds(0, t)])
        pltpu.sync_copy(tsh_hbm, tsh_v.at[pl.ds(0, t)])

        @pl.loop(0, _TABLE_PAD // _LANES)
        def _fuse(j):
            o = j * _LANES
            d = lut_v[pl.ds(o, _LANES)]
            d = jnp.minimum(jnp.maximum(d, 0), t - 1)
            fsc_v[pl.ds(o, _LANES)] = plsc.load_gather(tsc_v, [d])
            fsh_v[pl.ds(o, _LANES)] = plsc.load_gather(tsh_v, [d])

        out_flight = {}
        for c in range(_NCHUNK):
            b = c % 2
            if c + 1 < _NCHUNK:
                in_flight[c + 1] = fetch(c + 1)
            for h in in_flight.pop(c):
                h.wait()
            if c - 2 in out_flight:
                out_flight.pop(c - 2).wait()
            ev, av, ov = evs[b], avs[b], ovs[b]

            @plsc.parallel_loop(0, ch // _LANES, unroll=8)
            def _main(j):
                o = j * _LANES
                idx = ev[pl.ds(o, _LANES)]
                sc = plsc.load_gather(fsc_v, [idx])
                sh = plsc.load_gather(fsh_v, [idx])
                ov[pl.ds(o, _LANES)] = av[pl.ds(o, _LANES)] * sc + sh

            out_flight[c] = pltpu.async_copy(
                ov, out_hbm.at[pl.ds(base + c * ch, ch)], sos[b])
        for h in out_flight.values():
            h.wait()

    run = pl.kernel(
        body,
        out_type=jax.ShapeDtypeStruct((n_sc,), jnp.float32),
        mesh=mesh,
        compiler_params=pltpu.CompilerParams(needs_layout_passes=False),
        scratch_types=[
            pltpu.VMEM((ch,), jnp.int32),
            pltpu.VMEM((ch,), jnp.int32),
            pltpu.VMEM((ch,), jnp.float32),
            pltpu.VMEM((ch,), jnp.float32),
            pltpu.VMEM((ch,), jnp.float32),
            pltpu.VMEM((ch,), jnp.float32),
            pltpu.VMEM((_TABLE_PAD,), jnp.int32),
            pltpu.VMEM((_TABLE_PAD,), jnp.float32),
            pltpu.VMEM((_TABLE_PAD,), jnp.float32),
            pltpu.VMEM((_TABLE_PAD,), jnp.float32),
            pltpu.VMEM((_TABLE_PAD,), jnp.float32),
            pltpu.SemaphoreType.DMA,
            pltpu.SemaphoreType.DMA,
            pltpu.SemaphoreType.DMA,
            pltpu.SemaphoreType.DMA,
            pltpu.SemaphoreType.DMA,
            pltpu.SemaphoreType.DMA,
        ],
    )
    return run(elems, energy, lut, tsc, tsh)


def _tc_part(elems2, energy2, fsc, fsh, row0, n_rows):
    def body(fsc_ref, fsh_ref, e_ref, a_ref, o_ref):
        e = e_ref[...]
        a = a_ref[...]
        scb = jnp.broadcast_to(fsc_ref[...][None, :], e.shape)
        shb = jnp.broadcast_to(fsh_ref[...][None, :], e.shape)
        sc = jnp.take_along_axis(scb, e, axis=1)
        sh = jnp.take_along_axis(shb, e, axis=1)
        o_ref[...] = a * sc + sh

    blk0 = row0 // _TC_BLK_ROWS
    return pl.pallas_call(
        body,
        grid=(n_rows // _TC_BLK_ROWS,),
        in_specs=[
            pl.BlockSpec((_TABLE_PAD,), lambda i: (0,)),
            pl.BlockSpec((_TABLE_PAD,), lambda i: (0,)),
            pl.BlockSpec((_TC_BLK_ROWS, 128), lambda i: (i + blk0, 0)),
            pl.BlockSpec((_TC_BLK_ROWS, 128), lambda i: (i + blk0, 0)),
        ],
        out_specs=pl.BlockSpec((_TC_BLK_ROWS, 128), lambda i: (i, 0)),
        out_shape=jax.ShapeDtypeStruct((n_rows, 128), jnp.float32),
    )(fsc, fsh, elems2, energy2)


def kernel(elems, atomic_energy, scales, shifts, elem_lookup):
    n = elems.shape[0]
    lut = elem_lookup.astype(jnp.int32)
    n_sc = n // 2
    n_tc = n - n_sc
    assert n_sc % 128 == 0 and n_tc % (128 * _TC_BLK_ROWS) == 0
    e2 = elems.reshape(n // 128, 128)
    a2 = atomic_energy.reshape(n // 128, 128)
    t = lut.shape[0]
    fsc = jnp.pad(jnp.take(scales, lut, axis=0), (0, _TABLE_PAD - t))
    fsh = jnp.pad(jnp.take(shifts, lut, axis=0), (0, _TABLE_PAD - t))
    sc_out = _sc_part(elems, atomic_energy, lut, scales, shifts, n_sc)
    tc_out = _tc_part(e2, a2, fsc, fsh, n_sc // 128, n_tc // 128)
    return jnp.concatenate([sc_out, tc_out.reshape(n_tc)])


# speedup vs baseline: 1.0647x; 1.0647x over previous
import functools

import jax
import jax.numpy as jnp
from jax import lax
from jax.experimental import pallas as pl
from jax.experimental.pallas import tpu as pltpu
from jax.experimental.pallas import tpu_sc as plsc

_LANES = 16
_TABLE_PAD = 128
_NCHUNK = 4


def _scale_shift_sc(elems, energy, lut, tsc, tsh):
    n = elems.shape[0]
    mesh = plsc.VectorSubcoreMesh(core_axis_name="c", subcore_axis_name="s")
    n_workers = mesh.num_cores * mesh.num_subcores
    per_w = n // n_workers
    ch = per_w // _NCHUNK
    assert n % n_workers == 0 and per_w % (_NCHUNK * _LANES) == 0

    def body(elems_hbm, energy_hbm, lut_hbm, tsc_hbm, tsh_hbm, out_hbm,
             ev0, ev1, ev2, ev3, av0, av1, av2, av3,
             lut_v, tsc_v, tsh_v, fsc_v, fsh_v,
             se0, se1, se2, se3, sa0, sa1, sa2, sa3, so):
        wid = lax.axis_index("s") * mesh.num_cores + lax.axis_index("c")
        base = wid * per_w
        evs, avs = [ev0, ev1, ev2, ev3], [av0, av1, av2, av3]
        ses, sas = [se0, se1, se2, se3], [sa0, sa1, sa2, sa3]

        in_flight = []
        for c in range(_NCHUNK):
            src = pl.ds(base + c * ch, ch)
            in_flight.append(
                (pltpu.async_copy(elems_hbm.at[src], evs[c], ses[c]),
                 pltpu.async_copy(energy_hbm.at[src], avs[c], sas[c])))

        t = lut_hbm.shape[0]
        pltpu.sync_copy(lut_hbm, lut_v.at[pl.ds(0, t)])
        pltpu.sync_copy(tsc_hbm, tsc_v.at[pl.ds(0, t)])
        pltpu.sync_copy(tsh_hbm, tsh_v.at[pl.ds(0, t)])

        @pl.loop(0, _TABLE_PAD // _LANES)
        def _fuse(j):
            o = j * _LANES
            d = lut_v[pl.ds(o, _LANES)]
            d = jnp.minimum(jnp.maximum(d, 0), t - 1)
            fsc_v[pl.ds(o, _LANES)] = plsc.load_gather(tsc_v, [d])
            fsh_v[pl.ds(o, _LANES)] = plsc.load_gather(tsh_v, [d])

        out_flight = []
        for c in range(_NCHUNK):
            for h in in_flight[c]:
                h.wait()
            ev, av = evs[c], avs[c]

            @plsc.parallel_loop(0, ch // _LANES, unroll=8)
            def _main(j):
                o = j * _LANES
                idx = ev[pl.ds(o, _LANES)]
                sc = plsc.load_gather(fsc_v, [idx])
                sh = plsc.load_gather(fsh_v, [idx])
                av[pl.ds(o, _LANES)] = av[pl.ds(o, _LANES)] * sc + sh

            out_flight.append(pltpu.async_copy(
                av, out_hbm.at[pl.ds(base + c * ch, ch)], so))
        for h in out_flight:
            h.wait()

    run = pl.kernel(
        body,
        out_type=jax.ShapeDtypeStruct((n,), jnp.float32),
        mesh=mesh,
        compiler_params=pltpu.CompilerParams(needs_layout_passes=False),
        scratch_types=(
            [pltpu.VMEM((ch,), jnp.int32) for _ in range(_NCHUNK)]
            + [pltpu.VMEM((ch,), jnp.float32) for _ in range(_NCHUNK)]
            + [pltpu.VMEM((_TABLE_PAD,), jnp.int32)]
            + [pltpu.VMEM((_TABLE_PAD,), jnp.float32) for _ in range(4)]
            + [pltpu.SemaphoreType.DMA for _ in range(2 * _NCHUNK + 1)]
        ),
    )
    return run(elems, energy, lut, tsc, tsh)


def kernel(elems, atomic_energy, scales, shifts, elem_lookup):
    return _scale_shift_sc(elems, atomic_energy,
                           elem_lookup.astype(jnp.int32), scales, shifts)


# speedup vs baseline: 1.0667x; 1.0019x over previous
import functools

import jax
import jax.numpy as jnp
from jax import lax
from jax.experimental import pallas as pl
from jax.experimental.pallas import tpu as pltpu
from jax.experimental.pallas import tpu_sc as plsc

_LANES = 16
_TABLE_PAD = 128
_NCHUNK = 4


def _scale_shift_sc(elems, energy, lut, tsc, tsh):
    n = elems.shape[0]
    mesh = plsc.VectorSubcoreMesh(core_axis_name="c", subcore_axis_name="s")
    n_workers = mesh.num_cores * mesh.num_subcores
    per_w = n // n_workers
    ch = per_w // _NCHUNK
    assert n % n_workers == 0 and per_w % (_NCHUNK * _LANES) == 0

    def body(elems_hbm, energy_hbm, lut_hbm, tsc_hbm, tsh_hbm, out_hbm,
             ev0, ev1, ev2, ev3, av0, av1, av2, av3, ov0, ov1,
             lut_v, tsc_v, tsh_v, fsc_v, fsh_v,
             se0, se1, se2, se3, sa0, sa1, sa2, sa3, so):
        wid = lax.axis_index("s") * mesh.num_cores + lax.axis_index("c")
        base = wid * per_w
        evs, avs = [ev0, ev1, ev2, ev3], [av0, av1, av2, av3]
        ovs = [ov0, ov1]
        ses, sas = [se0, se1, se2, se3], [sa0, sa1, sa2, sa3]

        in_flight = []
        for c in range(_NCHUNK):
            src = pl.ds(base + c * ch, ch)
            in_flight.append(
                (pltpu.async_copy(elems_hbm.at[src], evs[c], ses[c]),
                 pltpu.async_copy(energy_hbm.at[src], avs[c], sas[c])))

        t = lut_hbm.shape[0]
        pltpu.sync_copy(lut_hbm, lut_v.at[pl.ds(0, t)])
        pltpu.sync_copy(tsc_hbm, tsc_v.at[pl.ds(0, t)])
        pltpu.sync_copy(tsh_hbm, tsh_v.at[pl.ds(0, t)])

        @pl.loop(0, _TABLE_PAD // _LANES)
        def _fuse(j):
            o = j * _LANES
            d = lut_v[pl.ds(o, _LANES)]
            d = jnp.minimum(jnp.maximum(d, 0), t - 1)
            fsc_v[pl.ds(o, _LANES)] = plsc.load_gather(tsc_v, [d])
            fsh_v[pl.ds(o, _LANES)] = plsc.load_gather(tsh_v, [d])

        out_flight = []
        for c in range(_NCHUNK):
            for h in in_flight[c]:
                h.wait()
            ev, av, ov = evs[c], avs[c], ovs[c % 2]
            if c >= 2:
                out_flight[c - 2].wait()

            @plsc.parallel_loop(0, ch // _LANES, unroll=8)
            def _main(j):
                o = j * _LANES
                idx = ev[pl.ds(o, _LANES)]
                sc = plsc.load_gather(fsc_v, [idx])
                sh = plsc.load_gather(fsh_v, [idx])
                ov[pl.ds(o, _LANES)] = av[pl.ds(o, _LANES)] * sc + sh

            out_flight.append(pltpu.async_copy(
                ov, out_hbm.at[pl.ds(base + c * ch, ch)], so))
        for h in out_flight[-2:]:
            h.wait()

    run = pl.kernel(
        body,
        out_type=jax.ShapeDtypeStruct((n,), jnp.float32),
        mesh=mesh,
        compiler_params=pltpu.CompilerParams(needs_layout_passes=False),
        scratch_types=(
            [pltpu.VMEM((ch,), jnp.int32) for _ in range(_NCHUNK)]
            + [pltpu.VMEM((ch,), jnp.float32) for _ in range(_NCHUNK)]
            + [pltpu.VMEM((ch,), jnp.float32) for _ in range(2)]
            + [pltpu.VMEM((_TABLE_PAD,), jnp.int32)]
            + [pltpu.VMEM((_TABLE_PAD,), jnp.float32) for _ in range(4)]
            + [pltpu.SemaphoreType.DMA for _ in range(2 * _NCHUNK + 1)]
        ),
    )
    return run(elems, energy, lut, tsc, tsh)


def kernel(elems, atomic_energy, scales, shifts, elem_lookup):
    return _scale_shift_sc(elems, atomic_energy,
                           elem_lookup.astype(jnp.int32), scales, shifts)


# speedup vs baseline: 1.1364x; 1.0654x over previous
import functools

import jax
import jax.numpy as jnp
from jax import lax
from jax.experimental import pallas as pl
from jax.experimental.pallas import tpu as pltpu
from jax.experimental.pallas import tpu_sc as plsc

_LANES = 16
_TABLE_PAD = 128


_NCHUNK = 4


def _scale_shift_sc(elems, energy, lut, tsc, tsh):
    n = elems.shape[0]
    mesh = plsc.VectorSubcoreMesh(core_axis_name="c", subcore_axis_name="s")
    n_workers = mesh.num_cores * mesh.num_subcores
    per_w = n // n_workers
    ch = per_w // _NCHUNK
    assert n % n_workers == 0 and per_w % (_NCHUNK * _LANES) == 0

    def body(elems_hbm, energy_hbm, lut_hbm, tsc_hbm, tsh_hbm, out_hbm,
             ev0, ev1, av0, av1, ov0, ov1,
             lut_v, tsc_v, tsh_v, fsc_v, fsh_v,
             se0, se1, sa0, sa1, so0, so1):
        wid = lax.axis_index("s") * mesh.num_cores + lax.axis_index("c")
        base = wid * per_w
        evs, avs, ovs = [ev0, ev1], [av0, av1], [ov0, ov1]
        ses, sas, sos = [se0, se1], [sa0, sa1], [so0, so1]

        def fetch(c):
            b = c % 2
            src = pl.ds(base + c * ch, ch)
            return (pltpu.async_copy(elems_hbm.at[src], evs[b], ses[b]),
                    pltpu.async_copy(energy_hbm.at[src], avs[b], sas[b]))

        in_flight = {0: fetch(0)}

        t = lut_hbm.shape[0]
        pltpu.sync_copy(lut_hbm, lut_v.at[pl.ds(0, t)])
        pltpu.sync_copy(tsc_hbm, tsc_v.at[pl.ds(0, t)])
        pltpu.sync_copy(tsh_hbm, tsh_v.at[pl.ds(0, t)])

        @pl.loop(0, _TABLE_PAD // _LANES)
        def _fuse(j):
            o = j * _LANES
            d = lut_v[pl.ds(o, _LANES)]
            d = jnp.minimum(jnp.maximum(d, 0), t - 1)
            fsc_v[pl.ds(o, _LANES)] = plsc.load_gather(tsc_v, [d])
            fsh_v[pl.ds(o, _LANES)] = plsc.load_gather(tsh_v, [d])

        out_flight = {}
        for c in range(_NCHUNK):
            b = c % 2
            if c + 1 < _NCHUNK:
                in_flight[c + 1] = fetch(c + 1)
            for h in in_flight.pop(c):
                h.wait()
            if c - 2 in out_flight:
                out_flight.pop(c - 2).wait()
            ev, av, ov = evs[b], avs[b], ovs[b]

            @plsc.parallel_loop(0, ch // _LANES, unroll=8)
            def _main(j):
                o = j * _LANES
                idx = ev[pl.ds(o, _LANES)]
                sc = plsc.load_gather(fsc_v, [idx])
                sh = plsc.load_gather(fsh_v, [idx])
                ov[pl.ds(o, _LANES)] = av[pl.ds(o, _LANES)] * sc + sh

            out_flight[c] = pltpu.async_copy(
                ov, out_hbm.at[pl.ds(base + c * ch, ch)], sos[b])
        for h in out_flight.values():
            h.wait()

    run = pl.kernel(
        body,
        out_type=jax.ShapeDtypeStruct((n,), jnp.float32),
        mesh=mesh,
        compiler_params=pltpu.CompilerParams(needs_layout_passes=False),
        scratch_types=[
            pltpu.VMEM((ch,), jnp.int32),
            pltpu.VMEM((ch,), jnp.int32),
            pltpu.VMEM((ch,), jnp.float32),
            pltpu.VMEM((ch,), jnp.float32),
            pltpu.VMEM((ch,), jnp.float32),
            pltpu.VMEM((ch,), jnp.float32),
            pltpu.VMEM((_TABLE_PAD,), jnp.int32),
            pltpu.VMEM((_TABLE_PAD,), jnp.float32),
            pltpu.VMEM((_TABLE_PAD,), jnp.float32),
            pltpu.VMEM((_TABLE_PAD,), jnp.float32),
            pltpu.VMEM((_TABLE_PAD,), jnp.float32),
            pltpu.SemaphoreType.DMA,
            pltpu.SemaphoreType.DMA,
            pltpu.SemaphoreType.DMA,
            pltpu.SemaphoreType.DMA,
            pltpu.SemaphoreType.DMA,
            pltpu.SemaphoreType.DMA,
        ],
    )
    return run(elems, energy, lut, tsc, tsh)


def kernel(elems, atomic_energy, scales, shifts, elem_lookup):
    return _scale_shift_sc(elems, atomic_energy,
                           elem_lookup.astype(jnp.int32), scales, shifts)
